# baseline (speedup 1.0000x reference)
import functools

import jax
import jax.numpy as jnp
from jax import lax
from jax.experimental import pallas as pl
from jax.experimental.pallas import tpu as pltpu
from jax.experimental.pallas import tpu_sc as plsc

_INFO = plsc.get_sparse_core_info()
_NC, _NS = _INFO.num_cores, _INFO.num_subcores
_NW = _NC * _NS


def _gather_body(b_per_w, table_hbm, idx_hbm, out_hbm, idx_v, rows_v, sem):
    wid = lax.axis_index("s") * _NC + lax.axis_index("c")
    base = wid * b_per_w
    pltpu.sync_copy(idx_hbm.at[pl.ds(base, b_per_w)], idx_v)
    pltpu.async_copy(table_hbm.at[idx_v], rows_v, sem).wait()
    pltpu.sync_copy(rows_v, out_hbm.at[pl.ds(base, b_per_w)])


def kernel(indices, table):
    B = indices.shape[0]
    V, D = table.shape
    b_per_w = B // _NW
    mesh = plsc.VectorSubcoreMesh(core_axis_name="c", subcore_axis_name="s")
    gather = pl.kernel(
        functools.partial(_gather_body, b_per_w),
        mesh=mesh,
        out_type=jax.ShapeDtypeStruct((B, D), jnp.float32),
        scratch_types=[
            pltpu.VMEM((b_per_w,), jnp.int32),
            pltpu.VMEM((b_per_w, D), jnp.float32),
            pltpu.SemaphoreType.DMA,
        ],
        compiler_params=pltpu.CompilerParams(use_tc_tiling_on_sc=False),
    )
    return gather(table, indices.astype(jnp.int32))


# speedup vs baseline: 1.0029x; 1.0029x over previous
import functools

import jax
import jax.numpy as jnp
from jax import lax
from jax.experimental import pallas as pl
from jax.experimental.pallas import tpu as pltpu
from jax.experimental.pallas import tpu_sc as plsc

_INFO = plsc.get_sparse_core_info()
_NC, _NS = _INFO.num_cores, _INFO.num_subcores
_NW = _NC * _NS


def _gather_body(b_per_w, table_hbm, idx_hbm, out_hbm, idx_v, rows_v, sem):
    wid = lax.axis_index("s") * _NC + lax.axis_index("c")
    base = wid * b_per_w
    pltpu.sync_copy(idx_hbm.at[pl.ds(base, b_per_w)], idx_v)
    pltpu.async_copy(table_hbm.at[idx_v], rows_v, sem).wait()
    pltpu.sync_copy(rows_v, out_hbm.at[pl.ds(base, b_per_w)])


def kernel(indices, table):
    B = indices.shape[0]
    V, D = table.shape
    b_per_w = B // _NW
    mesh = plsc.VectorSubcoreMesh(core_axis_name="c", subcore_axis_name="s")
    gather = pl.kernel(
        functools.partial(_gather_body, b_per_w),
        mesh=mesh,
        out_type=jax.ShapeDtypeStruct((B, D), jnp.float32),
        scratch_types=[
            pltpu.VMEM((b_per_w,), jnp.int32),
            pltpu.VMEM((b_per_w, D), jnp.float32),
            pltpu.SemaphoreType.DMA,
        ],
        compiler_params=pltpu.CompilerParams(use_tc_tiling_on_sc=False),
        cost_estimate=pl.CostEstimate(
            flops=0,
            bytes_accessed=2 * B * D * 4 + B * 4,
            transcendentals=0,
        ),
    )
    return gather(table, indices.astype(jnp.int32))


# speedup vs baseline: 1.6959x; 1.6911x over previous
import functools

import jax
import jax.numpy as jnp
from jax import lax
from jax.experimental import pallas as pl
from jax.experimental.pallas import tpu as pltpu
from jax.experimental.pallas import tpu_sc as plsc

_INFO = plsc.get_sparse_core_info()
_NC, _NS = _INFO.num_cores, _INFO.num_subcores
_NW = _NC * _NS

_CAP = 768
_WCAP = 768
_GCAP = 96
_BCAP = 32
_WIN = 16


def _compact_append(dst_i, dst_b, iv, bv, m, n):
    iota = jnp.arange(16, dtype=jnp.int32)
    neg1 = jnp.full((16,), -1, jnp.int32)
    keys = jnp.where(m, iota, iota + 16)
    si = plsc.sort_key_val(keys, jnp.where(m, iv, neg1))[-1]
    sb = plsc.sort_key_val(keys, jnp.where(m, bv, neg1))[-1]
    dst_i[pl.ds(n, 16)] = si
    dst_b[pl.ds(n, 16)] = sb
    cnt = plsc.all_reduce_population_count(m)
    return n + cnt[0]


def _sweep_body(B, V, NTC, NPW, tt_hbm, idx_hbm, stage_hbm, pos_hbm,
                idx_v, wl_b, wl_i, win_b, win_i, blk_b, blk_i,
                stage_v, pos_v, buf_v, sem, bsem):
    i32 = jnp.int32
    wid = lax.axis_index("s") * _NC + lax.axis_index("c")
    c0 = wid * NPW
    iota = jnp.arange(16, dtype=i32)
    lane0 = iota == 0

    pltpu.sync_copy(idx_hbm, idx_v)

    neg1 = jnp.full((16,), -1, i32)

    @pl.loop(0, _CAP, step=16)
    def _(k):
        pos_v[pl.ds(k, 16)] = neg1

    @pl.loop(0, _WCAP + 16, step=16)
    def _(k):
        wl_i[pl.ds(k, 16)] = neg1

    @pl.loop(0, B // 16, init_carry=jnp.asarray(0, i32))
    def n0(k, n):
        iv = idx_v[pl.ds(k * 16, 16)]
        cv = lax.shift_right_logical(iv, 7)
        m = (cv >= c0) & (cv < c0 + NPW)
        bv = iota + k * 16
        return _compact_append(wl_i, wl_b, iv, bv, m, n)

    del n0

    @pl.loop(0, NPW, init_carry=jnp.asarray(0, i32))
    def slot_final(bk, slot):
        c = c0 + bk
        cw0 = c0 + (bk & ~(_WIN - 1))

        @pl.when(bk & (_WIN - 1) == 0)
        def _():
            @pl.loop(0, (_GCAP + 16) // 16)
            def _(k):
                win_i[pl.ds(k * 16, 16)] = neg1

            @pl.loop(0, (_WCAP + 16) // 16, init_carry=jnp.asarray(0, i32))
            def _(k, n):
                iv = wl_i[pl.ds(k * 16, 16)]
                bv = wl_b[pl.ds(k * 16, 16)]
                cv = lax.shift_right_logical(iv, 7)
                m = (iv >= 0) & (cv >= cw0) & (cv < cw0 + _WIN)
                return _compact_append(win_i, win_b, iv, bv, m, n)

        cf = jnp.minimum(c, NTC - 2)

        @pl.when(c < NTC)
        def _():
            pltpu.sync_copy(
                tt_hbm.at[:, pl.ds(pl.multiple_of(cf * 128, 128), 128)],
                buf_v,
            )

        @pl.loop(0, (_GCAP + 16) // 16, init_carry=jnp.asarray(0, i32))
        def n2(k, n):
            iv = win_i[pl.ds(k * 16, 16)]
            bv = win_b[pl.ds(k * 16, 16)]
            cv = lax.shift_right_logical(iv, 7)
            m = (iv >= 0) & (cv == c)
            return _compact_append(blk_i, blk_b, iv, bv, m, n)

        @pl.loop(0, n2, init_carry=slot)
        def slot2(j, s):
            ivec = blk_i[pl.ds(j, 16)]
            bvec = blk_b[pl.ds(j, 16)]
            i = ivec[0]
            b = bvec[0]
            l = i - cf * 128
            lv = jnp.full((16,), l, i32)
            for c4 in range(4):
                dv = iota + c4 * 16
                vals = plsc.load_gather(buf_v, [dv, lv])
                stage_v[pl.ds(s * 64 + c4 * 16, 16)] = vals
            plsc.store_scatter(
                pos_v, [jnp.full((16,), s, i32)], jnp.full((16,), b, i32),
                mask=lane0,
            )
            return s + 1

        return slot2

    del slot_final

    pltpu.sync_copy(
        stage_v, stage_hbm.at[pl.ds(pl.multiple_of(wid * (_CAP * 64), 128),
                                    _CAP * 64)]
    )
    pltpu.sync_copy(
        pos_v, pos_hbm.at[pl.ds(pl.multiple_of(wid * _CAP, 128), _CAP)]
    )


def _scatter_body(B, D, stage_hbm, pos_hbm, out_hbm, st_v, pos_v, sem):
    wid = lax.axis_index("s") * _NC + lax.axis_index("c")
    pltpu.sync_copy(stage_hbm.at[pl.ds(wid * _CAP, _CAP)], st_v)
    pltpu.sync_copy(pos_hbm.at[pl.ds(wid * _CAP, _CAP)], pos_v)
    pltpu.async_copy(
        st_v, out_hbm.at[plsc.Indices(pos_v, ignored_value=-1)], sem
    ).wait()


def kernel(indices, table):
    B = indices.shape[0]
    V, D = table.shape
    NTC = (V + 127) // 128
    NPW = (NTC + _NW - 1) // _NW
    tt = table.T
    mesh = plsc.VectorSubcoreMesh(core_axis_name="c", subcore_axis_name="s")

    sweep = pl.kernel(
        functools.partial(_sweep_body, B, V, NTC, NPW),
        mesh=mesh,
        out_type=(
            jax.ShapeDtypeStruct((_NW * _CAP * 64,), jnp.float32),
            jax.ShapeDtypeStruct((_NW * _CAP,), jnp.int32),
        ),
        scratch_types=[
            pltpu.VMEM((B,), jnp.int32),
            pltpu.VMEM((_WCAP + 16,), jnp.int32),
            pltpu.VMEM((_WCAP + 16,), jnp.int32),
            pltpu.VMEM((_GCAP + 16,), jnp.int32),
            pltpu.VMEM((_GCAP + 16,), jnp.int32),
            pltpu.VMEM((_BCAP + 16,), jnp.int32),
            pltpu.VMEM((_BCAP + 16,), jnp.int32),
            pltpu.VMEM((_CAP * 64,), jnp.float32),
            pltpu.VMEM((_CAP,), jnp.int32),
            pltpu.VMEM((64, 128), jnp.float32),
            pltpu.SemaphoreType.DMA,
            pltpu.SemaphoreType.DMA,
        ],
        compiler_params=pltpu.CompilerParams(needs_layout_passes=False),
    )
    stage, pos = sweep(tt, indices.astype(jnp.int32))

    scatter = pl.kernel(
        functools.partial(_scatter_body, B, D),
        mesh=mesh,
        out_type=jax.ShapeDtypeStruct((B, D), jnp.float32),
        scratch_types=[
            pltpu.VMEM((_CAP, D), jnp.float32),
            pltpu.VMEM((_CAP,), jnp.int32),
            pltpu.SemaphoreType.DMA,
        ],
        compiler_params=pltpu.CompilerParams(use_tc_tiling_on_sc=False),
    )
    return scatter(stage.reshape(_NW * _CAP, 64), pos)


# speedup vs baseline: 3.7026x; 2.1832x over previous
import functools

import jax
import jax.numpy as jnp
from jax import lax
from jax.experimental import pallas as pl
from jax.experimental.pallas import tpu as pltpu
from jax.experimental.pallas import tpu_sc as plsc

_INFO = plsc.get_sparse_core_info()
_NC, _NS = _INFO.num_cores, _INFO.num_subcores
_NW = _NC * _NS

_CAP = 768
_WCAP = 768
_GCAP = 96
_BCAP = 32
_WIN = 16
_RING = 4


def _compact_append(dst_i, dst_b, iv, bv, m, n):
    iota = jnp.arange(16, dtype=jnp.int32)
    neg1 = jnp.full((16,), -1, jnp.int32)
    keys = jnp.where(m, iota, iota + 16)
    si = plsc.sort_key_val(keys, jnp.where(m, iv, neg1))[-1]
    sb = plsc.sort_key_val(keys, jnp.where(m, bv, neg1))[-1]
    dst_i[pl.ds(n, 16)] = si
    dst_b[pl.ds(n, 16)] = sb
    cnt = plsc.all_reduce_population_count(m)
    return n + cnt[0]


def _sweep_body(B, V, NTC, NPW, NPAD, tt_hbm, idx_hbm, stage_hbm, pos_hbm,
                idx_v, wl_b, wl_i, win_b, win_i, blk_b, blk_i,
                stage_v, pos_v, buf0, buf1, buf2, buf3,
                sem0, sem1, sem2, sem3):
    i32 = jnp.int32
    wid = lax.axis_index("s") * _NC + lax.axis_index("c")
    c0 = wid * NPW
    iota = jnp.arange(16, dtype=i32)
    lane0 = iota == 0
    bufs = (buf0, buf1, buf2, buf3)
    sems = (sem0, sem1, sem2, sem3)

    pltpu.sync_copy(idx_hbm, idx_v)

    neg1 = jnp.full((16,), -1, i32)

    @pl.loop(0, _CAP, step=16)
    def _(k):
        pos_v[pl.ds(k, 16)] = neg1

    @pl.loop(0, _WCAP + 16, step=16)
    def _(k):
        wl_i[pl.ds(k, 16)] = neg1

    @pl.loop(0, B // 16, init_carry=jnp.asarray(0, i32))
    def n0(k, n):
        iv = idx_v[pl.ds(k * 16, 16)]
        cv = lax.shift_right_logical(iv, 7)
        m = (cv >= c0) & (cv < c0 + NPW)
        bv = iota + k * 16
        return _compact_append(wl_i, wl_b, iv, bv, m, n)

    del n0

    def _fire(j, buf, sem):
        j = jnp.asarray(j, i32)
        c = c0 + j
        cf = jnp.minimum(c, NTC - 2)

        @pl.when((j < NPW) & (c < NTC))
        def _():
            pltpu.async_copy(
                tt_hbm.at[:, pl.ds(pl.multiple_of(cf * 128, 128), 128)],
                buf, sem,
            )

    for u in range(_RING):
        _fire(u, bufs[u], sems[u])

    @pl.loop(0, NPAD, step=_RING, init_carry=jnp.asarray(0, i32))
    def slot_final(bk, slot):
        for u in range(_RING):
            j = bk + u
            c = c0 + j
            cw0 = c0 + (j & ~(_WIN - 1))
            cf = jnp.minimum(c, NTC - 2)

            @pl.when(j & (_WIN - 1) == 0)
            def _():
                @pl.loop(0, (_GCAP + 16) // 16)
                def _(k):
                    win_i[pl.ds(k * 16, 16)] = neg1

                @pl.loop(0, (_WCAP + 16) // 16,
                         init_carry=jnp.asarray(0, i32))
                def _(k, n):
                    iv = wl_i[pl.ds(k * 16, 16)]
                    bv = wl_b[pl.ds(k * 16, 16)]
                    cv = lax.shift_right_logical(iv, 7)
                    m = (iv >= 0) & (cv >= cw0) & (cv < cw0 + _WIN)
                    return _compact_append(win_i, win_b, iv, bv, m, n)

            @pl.when((j < NPW) & (c < NTC))
            def _():
                pltpu.make_async_copy(
                    tt_hbm.at[:, pl.ds(0, 128)], bufs[u], sems[u]
                ).wait()

            @pl.loop(0, (_GCAP + 16) // 16, init_carry=jnp.asarray(0, i32))
            def n2(k, n):
                iv = win_i[pl.ds(k * 16, 16)]
                bv = win_b[pl.ds(k * 16, 16)]
                cv = lax.shift_right_logical(iv, 7)
                m = (iv >= 0) & (cv == c)
                return _compact_append(blk_i, blk_b, iv, bv, m, n)

            buf = bufs[u]

            @pl.loop(0, n2, init_carry=slot)
            def slot2(jj, s):
                ivec = blk_i[pl.ds(jj, 16)]
                bvec = blk_b[pl.ds(jj, 16)]
                i = ivec[0]
                b = bvec[0]
                l = i - cf * 128
                lv = jnp.full((16,), l, i32)
                for c4 in range(4):
                    dv = iota + c4 * 16
                    vals = plsc.load_gather(buf, [dv, lv])
                    stage_v[pl.ds(s * 64 + c4 * 16, 16)] = vals
                plsc.store_scatter(
                    pos_v, [jnp.full((16,), s, i32)],
                    jnp.full((16,), b, i32), mask=lane0,
                )
                return s + 1

            slot = slot2
            _fire(j + _RING, bufs[u], sems[u])
        return slot

    del slot_final

    pltpu.sync_copy(
        stage_v, stage_hbm.at[pl.ds(pl.multiple_of(wid * (_CAP * 64), 128),
                                    _CAP * 64)]
    )
    pltpu.sync_copy(
        pos_v, pos_hbm.at[pl.ds(pl.multiple_of(wid * _CAP, 128), _CAP)]
    )


def _scatter_body(B, D, stage_hbm, pos_hbm, out_hbm, st_v, pos_v, sem):
    wid = lax.axis_index("s") * _NC + lax.axis_index("c")
    pltpu.sync_copy(stage_hbm.at[pl.ds(wid * _CAP, _CAP)], st_v)
    pltpu.sync_copy(pos_hbm.at[pl.ds(wid * _CAP, _CAP)], pos_v)
    pltpu.async_copy(
        st_v, out_hbm.at[plsc.Indices(pos_v, ignored_value=-1)], sem
    ).wait()


def kernel(indices, table):
    B = indices.shape[0]
    V, D = table.shape
    NTC = (V + 127) // 128
    NPW = (NTC + _NW - 1) // _NW
    NPAD = -(-NPW // _RING) * _RING
    tt = table.T
    mesh = plsc.VectorSubcoreMesh(core_axis_name="c", subcore_axis_name="s")

    sweep = pl.kernel(
        functools.partial(_sweep_body, B, V, NTC, NPW, NPAD),
        mesh=mesh,
        out_type=(
            jax.ShapeDtypeStruct((_NW * _CAP * 64,), jnp.float32),
            jax.ShapeDtypeStruct((_NW * _CAP,), jnp.int32),
        ),
        scratch_types=[
            pltpu.VMEM((B,), jnp.int32),
            pltpu.VMEM((_WCAP + 16,), jnp.int32),
            pltpu.VMEM((_WCAP + 16,), jnp.int32),
            pltpu.VMEM((_GCAP + 16,), jnp.int32),
            pltpu.VMEM((_GCAP + 16,), jnp.int32),
            pltpu.VMEM((_BCAP + 16,), jnp.int32),
            pltpu.VMEM((_BCAP + 16,), jnp.int32),
            pltpu.VMEM((_CAP * 64,), jnp.float32),
            pltpu.VMEM((_CAP,), jnp.int32),
            pltpu.VMEM((64, 128), jnp.float32),
            pltpu.VMEM((64, 128), jnp.float32),
            pltpu.VMEM((64, 128), jnp.float32),
            pltpu.VMEM((64, 128), jnp.float32),
            pltpu.SemaphoreType.DMA,
            pltpu.SemaphoreType.DMA,
            pltpu.SemaphoreType.DMA,
            pltpu.SemaphoreType.DMA,
        ],
        compiler_params=pltpu.CompilerParams(needs_layout_passes=False),
    )
    stage, pos = sweep(tt, indices.astype(jnp.int32))

    scatter = pl.kernel(
        functools.partial(_scatter_body, B, D),
        mesh=mesh,
        out_type=jax.ShapeDtypeStruct((B, D), jnp.float32),
        scratch_types=[
            pltpu.VMEM((_CAP, D), jnp.float32),
            pltpu.VMEM((_CAP,), jnp.int32),
            pltpu.SemaphoreType.DMA,
        ],
        compiler_params=pltpu.CompilerParams(use_tc_tiling_on_sc=False),
    )
    return scatter(stage.reshape(_NW * _CAP, 64), pos)


# speedup vs baseline: 3.7090x; 1.0017x over previous
import functools

import jax
import jax.numpy as jnp
from jax import lax
from jax.experimental import pallas as pl
from jax.experimental.pallas import tpu as pltpu
from jax.experimental.pallas import tpu_sc as plsc

_INFO = plsc.get_sparse_core_info()
_NC, _NS = _INFO.num_cores, _INFO.num_subcores
_NW = _NC * _NS

_CAP = 768
_WCAP = 768
_GCAP = 96
_BCAP = 32
_WIN = 16
_RING = 4


def _compact_append(dst_i, dst_b, iv, bv, m, n):
    iota = jnp.arange(16, dtype=jnp.int32)
    neg1 = jnp.full((16,), -1, jnp.int32)
    keys = jnp.where(m, iota, iota + 16)
    si = plsc.sort_key_val(keys, jnp.where(m, iv, neg1))[-1]
    sb = plsc.sort_key_val(keys, jnp.where(m, bv, neg1))[-1]
    dst_i[pl.ds(n, 16)] = si
    dst_b[pl.ds(n, 16)] = sb
    cnt = plsc.all_reduce_population_count(m)
    return n + cnt[0]


def _sweep_body(B, V, NTC, NPW, NPAD, tt_hbm, idx_hbm, stage_hbm, pos_hbm,
                idx_v, wl_b, wl_i, win_b, win_i, blk_b, blk_i,
                stage_v, pos_v, buf0, buf1, buf2, buf3,
                sem0, sem1, sem2, sem3):
    i32 = jnp.int32
    wid = lax.axis_index("s") * _NC + lax.axis_index("c")
    c0 = wid * NPW
    iota = jnp.arange(16, dtype=i32)
    lane0 = iota == 0
    bufs = (buf0, buf1, buf2, buf3)
    sems = (sem0, sem1, sem2, sem3)

    pltpu.sync_copy(idx_hbm, idx_v)

    neg1 = jnp.full((16,), -1, i32)

    @pl.loop(0, _CAP, step=16)
    def _(k):
        pos_v[pl.ds(k, 16)] = neg1

    @pl.loop(0, _WCAP + 16, step=16)
    def _(k):
        wl_i[pl.ds(k, 16)] = neg1

    @pl.loop(0, B // 16, init_carry=jnp.asarray(0, i32))
    def n0(k, n):
        iv = idx_v[pl.ds(k * 16, 16)]
        cv = lax.shift_right_logical(iv, 7)
        m = (cv >= c0) & (cv < c0 + NPW)
        bv = iota + k * 16
        return _compact_append(wl_i, wl_b, iv, bv, m, n)

    del n0

    def _fire(j, buf, sem):
        j = jnp.asarray(j, i32)
        c = c0 + j
        cf = c

        @pl.when((j < NPW) & (c < NTC))
        def _():
            pltpu.async_copy(
                tt_hbm.at[:, pl.ds(pl.multiple_of(cf * 128, 128), 128)],
                buf, sem,
            )

    for u in range(_RING):
        _fire(u, bufs[u], sems[u])

    @pl.loop(0, NPAD, step=_RING, init_carry=jnp.asarray(0, i32))
    def slot_final(bk, slot):
        for u in range(_RING):
            j = bk + u
            c = c0 + j
            cw0 = c0 + (j & ~(_WIN - 1))
            cf = c

            @pl.when(j & (_WIN - 1) == 0)
            def _():
                @pl.loop(0, (_GCAP + 16) // 16)
                def _(k):
                    win_i[pl.ds(k * 16, 16)] = neg1

                @pl.loop(0, (_WCAP + 16) // 16,
                         init_carry=jnp.asarray(0, i32))
                def _(k, n):
                    iv = wl_i[pl.ds(k * 16, 16)]
                    bv = wl_b[pl.ds(k * 16, 16)]
                    cv = lax.shift_right_logical(iv, 7)
                    m = (iv >= 0) & (cv >= cw0) & (cv < cw0 + _WIN)
                    return _compact_append(win_i, win_b, iv, bv, m, n)

            @pl.when((j < NPW) & (c < NTC))
            def _():
                pltpu.make_async_copy(
                    tt_hbm.at[:, pl.ds(0, 128)], bufs[u], sems[u]
                ).wait()

            @pl.loop(0, (_GCAP + 16) // 16, init_carry=jnp.asarray(0, i32))
            def n2(k, n):
                iv = win_i[pl.ds(k * 16, 16)]
                bv = win_b[pl.ds(k * 16, 16)]
                cv = lax.shift_right_logical(iv, 7)
                m = (iv >= 0) & (cv == c)
                return _compact_append(blk_i, blk_b, iv, bv, m, n)

            buf = bufs[u]

            @pl.loop(0, n2, init_carry=slot)
            def slot2(jj, s):
                ivec = blk_i[pl.ds(jj, 16)]
                bvec = blk_b[pl.ds(jj, 16)]
                i = ivec[0]
                b = bvec[0]
                l = i - cf * 128
                lv = jnp.full((16,), l, i32)
                for c4 in range(4):
                    dv = iota + c4 * 16
                    vals = plsc.load_gather(buf, [dv, lv])
                    stage_v[pl.ds(s * 64 + c4 * 16, 16)] = vals
                plsc.store_scatter(
                    pos_v, [jnp.full((16,), s, i32)],
                    jnp.full((16,), b, i32), mask=lane0,
                )
                return s + 1

            slot = slot2
            _fire(j + _RING, bufs[u], sems[u])
        return slot

    del slot_final

    pltpu.sync_copy(
        stage_v, stage_hbm.at[pl.ds(pl.multiple_of(wid * (_CAP * 64), 128),
                                    _CAP * 64)]
    )
    pltpu.sync_copy(
        pos_v, pos_hbm.at[pl.ds(pl.multiple_of(wid * _CAP, 128), _CAP)]
    )


def _scatter_body(B, D, stage_hbm, pos_hbm, out_hbm, st_v, pos_v, sem):
    wid = lax.axis_index("s") * _NC + lax.axis_index("c")
    pltpu.sync_copy(stage_hbm.at[pl.ds(wid * _CAP, _CAP)], st_v)
    pltpu.sync_copy(pos_hbm.at[pl.ds(wid * _CAP, _CAP)], pos_v)
    pltpu.async_copy(
        st_v, out_hbm.at[plsc.Indices(pos_v, ignored_value=-1)], sem
    ).wait()


def kernel(indices, table):
    B = indices.shape[0]
    V, D = table.shape
    NTC = (V + 127) // 128
    NPW = (NTC + _NW - 1) // _NW
    NPAD = -(-NPW // _RING) * _RING
    tt = table.T
    mesh = plsc.VectorSubcoreMesh(core_axis_name="c", subcore_axis_name="s")

    sweep = pl.kernel(
        functools.partial(_sweep_body, B, V, NTC, NPW, NPAD),
        mesh=mesh,
        out_type=(
            jax.ShapeDtypeStruct((_NW * _CAP * 64,), jnp.float32),
            jax.ShapeDtypeStruct((_NW * _CAP,), jnp.int32),
        ),
        scratch_types=[
            pltpu.VMEM((B,), jnp.int32),
            pltpu.VMEM((_WCAP + 16,), jnp.int32),
            pltpu.VMEM((_WCAP + 16,), jnp.int32),
            pltpu.VMEM((_GCAP + 16,), jnp.int32),
            pltpu.VMEM((_GCAP + 16,), jnp.int32),
            pltpu.VMEM((_BCAP + 16,), jnp.int32),
            pltpu.VMEM((_BCAP + 16,), jnp.int32),
            pltpu.VMEM((_CAP * 64,), jnp.float32),
            pltpu.VMEM((_CAP,), jnp.int32),
            pltpu.VMEM((64, 128), jnp.float32),
            pltpu.VMEM((64, 128), jnp.float32),
            pltpu.VMEM((64, 128), jnp.float32),
            pltpu.VMEM((64, 128), jnp.float32),
            pltpu.SemaphoreType.DMA,
            pltpu.SemaphoreType.DMA,
            pltpu.SemaphoreType.DMA,
            pltpu.SemaphoreType.DMA,
        ],
        compiler_params=pltpu.CompilerParams(needs_layout_passes=False),
    )
    stage, pos = sweep(tt, indices.astype(jnp.int32))

    scatter = pl.kernel(
        functools.partial(_scatter_body, B, D),
        mesh=mesh,
        out_type=jax.ShapeDtypeStruct((B, D), jnp.float32),
        scratch_types=[
            pltpu.VMEM((_CAP, D), jnp.float32),
            pltpu.VMEM((_CAP,), jnp.int32),
            pltpu.SemaphoreType.DMA,
        ],
        compiler_params=pltpu.CompilerParams(use_tc_tiling_on_sc=False),
    )
    return scatter(stage.reshape(_NW * _CAP, 64), pos)
